# Initial kernel scaffold; baseline (speedup 1.0000x reference)
import jax
import jax.numpy as jnp
from jax.experimental import pallas as pl


def kernel(matrix, drop_param, gamma, drop_ratio):
    raise NotImplementedError("write your pallas kernel here")


# speedup vs baseline: 229.2501x; 229.2501x over previous
import jax
import jax.numpy as jnp
from jax.experimental import pallas as pl
from jax.experimental.pallas import tpu as pltpu

N = 4096
M = N * N
K = int(M * 0.2)

R = 128
T = N // R
R2 = 512
T2 = N // R2
R3 = 256
T3 = N // R3
ROUNDS = 12
SCAP = 4

_G_CACHE = []


def _gumbel_const():
    if not _G_CACHE:
        g = jax.random.gumbel(jax.random.key(42), (M,), jnp.float32)
        _G_CACHE.append(g.reshape(N, N))
    return _G_CACHE[0]


def _zero_body(m_ref, cnt_ref):
    cnt_ref[pl.program_id(0)] = jnp.sum((m_ref[...] == 0.0).astype(jnp.int32))


def _score_body(cz_ref, ab_ref, m_ref, ga_ref, gp_ref, s_ref, mx_ref, mn_ref,
                gs_ref):
    i = pl.program_id(0)
    v = m_ref[...]
    dp = ab_ref[0]
    gm = ab_ref[1]
    a = dp - (dp + gm) * v
    ls = jnp.minimum(a, 0.0) - jnp.log(1.0 + jnp.exp(-jnp.abs(a)))

    z = (v == 0.0)
    nloc = jnp.sum(z.astype(jnp.int32))
    c0 = cz_ref[i]

    rowid = jax.lax.broadcasted_iota(jnp.int32, (R, N), 0)
    colid = jax.lax.broadcasted_iota(jnp.int32, (R, N), 1)
    flat = rowid * N + colid

    big = jnp.int32(1 << 30)
    m = jnp.where(z, flat, big)
    lb = jnp.zeros((R, N), jnp.int32)
    for _ in range(SCAP):
        zp = jnp.min(m)
        lb = lb + (flat > zp).astype(jnp.int32)
        m = jnp.where(m == zp, big, m)

    def shifted(sig):
        rl = pltpu.roll(ga_ref[...], sig, axis=1)
        pl_row = pltpu.roll(gp_ref[7:8, :], sig, axis=1)
        up = jnp.concatenate([pl_row, rl[:-1]], axis=0)
        return jnp.where(colid >= sig, rl, up)

    gs_ref[...] = shifted(c0)
    for s in range(1, SCAP + 1):
        @pl.when(nloc >= s)
        def _(s=s):
            gs_ref[...] = jnp.where(lb == s, shifted(c0 + s), gs_ref[...])

    score = ls + gs_ref[...]
    neg_inf = jnp.float32(-jnp.inf)
    score = jnp.where(z, neg_inf, score)
    s_ref[...] = score
    mx_ref[i] = jnp.max(score)
    mn_ref[i] = jnp.min(jnp.where(z, jnp.float32(jnp.inf), score))


def _sel_body(bnd_ref, u_ref, t_ref, f_ref):
    r = pl.program_id(0)
    i = pl.program_id(1)
    kf = jnp.float32(K)

    @pl.when((r == 0) & (i == 0))
    def _init():
        f_ref[0] = bnd_ref[0]
        f_ref[1] = bnd_ref[1]
        f_ref[2] = bnd_ref[2]
        f_ref[3] = 0.0
        f_ref[5] = bnd_ref[1]
        f_ref[6] = jnp.float32(1e30)

    @pl.when(i == 0)
    def _round_start():
        lo = f_ref[0]
        hi = f_ref[1]
        clo = f_ref[2]
        chi = f_ref[3]
        denom = clo - chi
        frac = jnp.where(denom > 0.0, (clo - kf) / denom, jnp.float32(0.5))
        f_ref[4] = lo + (hi - lo) * jnp.clip(frac, 0.03, 0.97)
        f_ref[7] = 0.0

    t = f_ref[4]
    f_ref[7] = f_ref[7] + jnp.sum((u_ref[...] > t).astype(jnp.float32))

    @pl.when(i == pl.num_programs(1) - 1)
    def _round_end():
        c = f_ref[7]
        err = jnp.abs(c - kf)

        @pl.when(err < f_ref[6])
        def _():
            f_ref[6] = err
            f_ref[5] = f_ref[4]

        @pl.when(c > kf)
        def _():
            f_ref[0] = f_ref[4]
            f_ref[2] = c

        @pl.when(c <= kf)
        def _():
            f_ref[1] = f_ref[4]
            f_ref[3] = c

        t_ref[0] = f_ref[5]


def _out_body(t_ref, m_ref, u_ref, o_ref):
    t = t_ref[0]
    o_ref[...] = jnp.where(u_ref[...] > t, 0.0, m_ref[...])


def kernel(matrix, drop_param, gamma, drop_ratio):
    del drop_ratio

    zc = pl.pallas_call(
        _zero_body,
        grid=(T,),
        in_specs=[pl.BlockSpec((R, N), lambda i: (i, 0))],
        out_specs=pl.BlockSpec(memory_space=pltpu.SMEM),
        out_shape=jax.ShapeDtypeStruct((T,), jnp.int32),
    )(matrix)
    cz = (jnp.cumsum(zc) - zc).astype(jnp.int32)
    ab = jnp.concatenate([drop_param, gamma]).astype(jnp.float32)

    scores, mx, mn = pl.pallas_call(
        _score_body,
        grid=(T,),
        in_specs=[
            pl.BlockSpec(memory_space=pltpu.SMEM),
            pl.BlockSpec(memory_space=pltpu.SMEM),
            pl.BlockSpec((R, N), lambda i: (i, 0)),
            pl.BlockSpec((R, N), lambda i: (i, 0)),
            pl.BlockSpec((8, N), lambda i: (jnp.maximum(i * (R // 8) - 1, 0), 0)),
        ],
        out_specs=[
            pl.BlockSpec((R, N), lambda i: (i, 0)),
            pl.BlockSpec(memory_space=pltpu.SMEM),
            pl.BlockSpec(memory_space=pltpu.SMEM),
        ],
        out_shape=[
            jax.ShapeDtypeStruct((N, N), jnp.float32),
            jax.ShapeDtypeStruct((T,), jnp.float32),
            jax.ShapeDtypeStruct((T,), jnp.float32),
        ],
        scratch_shapes=[pltpu.VMEM((R, N), jnp.float32)],
    )(cz, ab, matrix, _gumbel_const(), _gumbel_const())

    meff = jnp.float32(M) - jnp.sum(zc).astype(jnp.float32)
    bnd = jnp.stack([jnp.min(mn), jnp.max(mx), meff, jnp.float32(0.0)])

    thr = pl.pallas_call(
        _sel_body,
        grid=(ROUNDS, T2),
        in_specs=[
            pl.BlockSpec(memory_space=pltpu.SMEM),
            pl.BlockSpec((R2, N), lambda r, i: (i, 0)),
        ],
        out_specs=pl.BlockSpec(memory_space=pltpu.SMEM),
        out_shape=jax.ShapeDtypeStruct((1,), jnp.float32),
        scratch_shapes=[pltpu.SMEM((8,), jnp.float32)],
    )(bnd, scores)

    out = pl.pallas_call(
        _out_body,
        grid=(T3,),
        in_specs=[
            pl.BlockSpec(memory_space=pltpu.SMEM),
            pl.BlockSpec((R3, N), lambda i: (i, 0)),
            pl.BlockSpec((R3, N), lambda i: (i, 0)),
        ],
        out_specs=pl.BlockSpec((R3, N), lambda i: (i, 0)),
        out_shape=jax.ShapeDtypeStruct((N, N), jnp.float32),
    )(thr, matrix, scores)
    return out


# speedup vs baseline: 253.0146x; 1.1037x over previous
import jax
import jax.numpy as jnp
from jax.experimental import pallas as pl
from jax.experimental.pallas import tpu as pltpu

N = 4096
M = N * N
K = int(M * 0.2)

R = 128
T = N // R
R2 = 512
T2 = N // R2
R3 = 256
T3 = N // R3
ROUNDS = 8
SCAP = 4

_G_CACHE = []


def _gumbel_const():
    if not _G_CACHE:
        g = jax.random.gumbel(jax.random.key(42), (M,), jnp.float32)
        _G_CACHE.append(g.reshape(N, N))
    return _G_CACHE[0]


def _zero_body(m_ref, cnt_ref):
    c = jnp.sum((m_ref[...] == 0.0).astype(jnp.int32))
    cnt_ref[...] = jnp.broadcast_to(c, (1, 8, 128))


def _score_body(cz_ref, ab_ref, m_ref, ga_ref, gp_ref, s_ref, mx_ref, mn_ref,
                s1_ref, gs_ref):
    i = pl.program_id(0)
    v = m_ref[...]
    dp = ab_ref[0]
    gm = ab_ref[1]
    a = dp - (dp + gm) * v
    ls = jnp.minimum(a, 0.0) - jnp.log(1.0 + jnp.exp(-jnp.abs(a)))

    z = (v == 0.0)
    nloc = jnp.sum(z.astype(jnp.int32))
    c0 = cz_ref[i]

    rowid = jax.lax.broadcasted_iota(jnp.int32, (R, N), 0)
    colid = jax.lax.broadcasted_iota(jnp.int32, (R, N), 1)
    flat = rowid * N + colid

    big = jnp.int32(1 << 30)
    m = jnp.where(z, flat, big)
    lb = jnp.zeros((R, N), jnp.int32)
    for _ in range(SCAP):
        zp = jnp.min(m)
        lb = lb + (flat > zp).astype(jnp.int32)
        m = jnp.where(m == zp, big, m)

    def shifted(sig):
        rl = pltpu.roll(ga_ref[...], sig, axis=1)
        pl_row = pltpu.roll(gp_ref[7:8, :], sig, axis=1)
        up = jnp.concatenate([pl_row, rl[:-1]], axis=0)
        return jnp.where(colid >= sig, rl, up)

    gs_ref[...] = shifted(c0)
    for s in range(1, SCAP + 1):
        @pl.when(nloc >= s)
        def _(s=s):
            gs_ref[...] = jnp.where(lb == s, shifted(c0 + s), gs_ref[...])

    score = ls + gs_ref[...]
    neg_inf = jnp.float32(-jnp.inf)
    score = jnp.where(z, neg_inf, score)
    s_ref[...] = score
    mx_ref[...] = jnp.broadcast_to(jnp.max(score), (1, 8, 128))
    mn_ref[...] = jnp.broadcast_to(
        jnp.min(jnp.where(z, jnp.float32(jnp.inf), score)), (1, 8, 128))
    s1_ref[...] = jnp.broadcast_to(jnp.sum(jnp.exp(ls)), (1, 8, 128))


def _sel_body(bnd_ref, u_ref, t_ref, f_ref):
    r = pl.program_id(0)
    i = pl.program_id(1)
    kf = jnp.float32(K)

    @pl.when((r == 0) & (i == 0))
    def _init():
        f_ref[0] = bnd_ref[0]
        f_ref[1] = bnd_ref[1]
        f_ref[2] = bnd_ref[2]
        f_ref[3] = 0.0
        f_ref[5] = bnd_ref[1]
        f_ref[6] = jnp.float32(1e30)

    @pl.when(i == 0)
    def _round_start():
        lo = f_ref[0]
        hi = f_ref[1]
        clo = f_ref[2]
        chi = f_ref[3]
        denom = clo - chi
        frac = jnp.where(denom > 0.0, (clo - kf) / denom, jnp.float32(0.5))
        t = lo + (hi - lo) * jnp.clip(frac, 0.001, 0.999)
        t = jnp.where(r == 0, jnp.clip(bnd_ref[3], lo, hi), t)
        f_ref[4] = t
        f_ref[7] = 0.0

    t = f_ref[4]
    f_ref[7] = f_ref[7] + jnp.sum((u_ref[...] > t).astype(jnp.float32))

    @pl.when(i == pl.num_programs(1) - 1)
    def _round_end():
        c = f_ref[7]
        err = jnp.abs(c - kf)

        @pl.when(err < f_ref[6])
        def _():
            f_ref[6] = err
            f_ref[5] = f_ref[4]

        @pl.when(c > kf)
        def _():
            f_ref[0] = f_ref[4]
            f_ref[2] = c

        @pl.when(c <= kf)
        def _():
            f_ref[1] = f_ref[4]
            f_ref[3] = c

        t_ref[0] = f_ref[5]


def _out_body(t_ref, m_ref, u_ref, o_ref):
    t = t_ref[0]
    o_ref[...] = jnp.where(u_ref[...] > t, 0.0, m_ref[...])


def kernel(matrix, drop_param, gamma, drop_ratio):
    del drop_ratio

    zc = pl.pallas_call(
        _zero_body,
        grid=(T,),
        in_specs=[pl.BlockSpec((R, N), lambda i: (i, 0))],
        out_specs=pl.BlockSpec((1, 8, 128), lambda i: (i, 0, 0)),
        out_shape=jax.ShapeDtypeStruct((T, 8, 128), jnp.int32),
        compiler_params=pltpu.CompilerParams(
            dimension_semantics=("parallel",)),
    )(matrix)[:, 0, 0]
    cz = (jnp.cumsum(zc) - zc).astype(jnp.int32)
    ab = jnp.concatenate([drop_param, gamma]).astype(jnp.float32)

    scores, mx, mn, s1 = pl.pallas_call(
        _score_body,
        grid=(T,),
        in_specs=[
            pl.BlockSpec(memory_space=pltpu.SMEM),
            pl.BlockSpec(memory_space=pltpu.SMEM),
            pl.BlockSpec((R, N), lambda i: (i, 0)),
            pl.BlockSpec((R, N), lambda i: (i, 0)),
            pl.BlockSpec((8, N), lambda i: (jnp.maximum(i * (R // 8) - 1, 0), 0)),
        ],
        out_specs=[
            pl.BlockSpec((R, N), lambda i: (i, 0)),
            pl.BlockSpec((1, 8, 128), lambda i: (i, 0, 0)),
            pl.BlockSpec((1, 8, 128), lambda i: (i, 0, 0)),
            pl.BlockSpec((1, 8, 128), lambda i: (i, 0, 0)),
        ],
        out_shape=[
            jax.ShapeDtypeStruct((N, N), jnp.float32),
            jax.ShapeDtypeStruct((T, 8, 128), jnp.float32),
            jax.ShapeDtypeStruct((T, 8, 128), jnp.float32),
            jax.ShapeDtypeStruct((T, 8, 128), jnp.float32),
        ],
        scratch_shapes=[pltpu.VMEM((R, N), jnp.float32)],
        compiler_params=pltpu.CompilerParams(
            dimension_semantics=("parallel",)),
    )(cz, ab, matrix, _gumbel_const(), _gumbel_const())

    meff = jnp.float32(M) - jnp.sum(zc).astype(jnp.float32)
    t0 = jnp.log(jnp.maximum(jnp.sum(s1[:, 0, 0]), 1e-30) / jnp.float32(K))
    bnd = jnp.stack([jnp.min(mn[:, 0, 0]), jnp.max(mx[:, 0, 0]), meff, t0])

    thr = pl.pallas_call(
        _sel_body,
        grid=(ROUNDS, T2),
        in_specs=[
            pl.BlockSpec(memory_space=pltpu.SMEM),
            pl.BlockSpec((R2, N), lambda r, i: (i, 0)),
        ],
        out_specs=pl.BlockSpec(memory_space=pltpu.SMEM),
        out_shape=jax.ShapeDtypeStruct((1,), jnp.float32),
        scratch_shapes=[pltpu.SMEM((8,), jnp.float32)],
    )(bnd, scores)

    out = pl.pallas_call(
        _out_body,
        grid=(T3,),
        in_specs=[
            pl.BlockSpec(memory_space=pltpu.SMEM),
            pl.BlockSpec((R3, N), lambda i: (i, 0)),
            pl.BlockSpec((R3, N), lambda i: (i, 0)),
        ],
        out_specs=pl.BlockSpec((R3, N), lambda i: (i, 0)),
        out_shape=jax.ShapeDtypeStruct((N, N), jnp.float32),
        compiler_params=pltpu.CompilerParams(
            dimension_semantics=("parallel",)),
    )(thr, matrix, scores)
    return out
